# baseline (speedup 1.0000x reference)
import functools

import jax
import jax.numpy as jnp
from jax import lax
from jax.experimental import pallas as pl
from jax.experimental.pallas import tpu as pltpu
from jax.experimental.pallas import tpu_sc as plsc

B = 4
N = 2048
CH = 256
DIM = 256
K = 16
K2 = 8
BN = B * N
R = B * N * K
F32 = jnp.float32


def _mmT(a, w):
    return lax.dot_general(a, w, (((1,), (1,)), ((), ())),
                           preferred_element_type=F32)


BF = jnp.bfloat16


def _mmT_bf(a, w):
    return lax.dot_general(a.astype(BF), w.astype(BF), (((1,), (1,)), ((), ())),
                           preferred_element_type=F32)


def _batcher_pairs(n):
    pairs = []
    p = 1
    while p < n:
        k = p
        while k >= 1:
            for j in range(k % p, n - k, 2 * k):
                for i in range(0, min(k, n - j - k)):
                    if (i + j) // (2 * p) == (i + j + k) // (2 * p):
                        pairs.append((i + j, i + j + k))
            k //= 2
        p *= 2
    return pairs

_SORT16 = _batcher_pairs(16)


def _k1_body(xyz_ref, pts_ref, wq_ref, bq_ref, wk_ref, bk_ref, wv_ref, bv_ref,
             pt_ref, qt_ref, ktx_ref, vt_ref, xt_ref, gidx_ref):
    b = pl.program_id(0)
    p = pts_ref[0]
    p_t = p.T
    pt_ref[...] = p_t
    qt_ref[...] = _mmT_bf(p_t, wq_ref[...]) + bq_ref[...][None, :]
    k_t = _mmT_bf(p_t, wk_ref[...]) + bk_ref[...][None, :]
    vt_ref[...] = _mmT_bf(p_t, wv_ref[...]) + bv_ref[...][None, :]

    x = xyz_ref[0]
    x16 = jnp.concatenate([x, jnp.zeros((13, N), F32)], axis=0)
    x_t = x16.T
    xt_ref[...] = x_t
    ktx_ref[...] = jnp.zeros((N, 384), F32)
    ktx_ref[:, 0:DIM] = k_t
    ktx_ref[:, DIM:DIM + 16] = x_t
    sq = jnp.sum(x_t * x_t, axis=1)

    T = 512
    for t in range(N // T):
        xt_tile = x_t[t * T:(t + 1) * T]
        g = _mmT(xt_tile, x_t)
        d = sq[t * T:(t + 1) * T][:, None] + sq[None, :] - 2.0 * g
        iota = lax.broadcasted_iota(jnp.int32, (T, N), 1)
        idxs = []
        for _ in range(K):
            m = jnp.min(d, axis=1, keepdims=True)
            am = jnp.min(jnp.where(d == m, iota, N), axis=1)
            idxs.append(am)
            d = jnp.where(iota == am[:, None], jnp.inf, d)
        for (i, j) in _SORT16:
            lo = jnp.minimum(idxs[i], idxs[j])
            hi = jnp.maximum(idxs[i], idxs[j])
            idxs[i], idxs[j] = lo, hi
        tile_idx = jnp.stack(idxs, axis=1) + b * N
        gidx_ref[pl.ds(t * T, T), :] = tile_idx


def _k1(xyz, points, wq, bq, wk, bk, wv, bv):
    full = lambda s: pl.BlockSpec(s, lambda b: (0,) * len(s))
    flat = lambda c: pl.BlockSpec((N, c), lambda b: (b, 0))
    return pl.pallas_call(
        _k1_body,
        grid=(B,),
        in_specs=[
            pl.BlockSpec((1, 3, N), lambda b: (b, 0, 0)),
            pl.BlockSpec((1, CH, N), lambda b: (b, 0, 0)),
            full((DIM, CH)), full((DIM,)),
            full((DIM, CH)), full((DIM,)),
            full((DIM, CH)), full((DIM,)),
        ],
        out_specs=[flat(CH), flat(DIM), flat(384), flat(DIM), flat(16),
                   flat(K)],
        out_shape=[
            jax.ShapeDtypeStruct((BN, CH), F32),
            jax.ShapeDtypeStruct((BN, DIM), F32),
            jax.ShapeDtypeStruct((BN, 384), F32),
            jax.ShapeDtypeStruct((BN, DIM), F32),
            jax.ShapeDtypeStruct((BN, 16), F32),
            jax.ShapeDtypeStruct((BN, K), jnp.int32),
        ],
    )(xyz, points, wq, bq, wk, bk, wv, bv)


_NC = 2
_NS = 16
_NW = _NC * _NS
_RPW = R // _NW
_CHUNK = 128
_NCHUNK = _RPW // _CHUNK


def _k2_sc_gather(gidx, ktx):
    mesh = plsc.VectorSubcoreMesh(core_axis_name="c", subcore_axis_name="s")

    @functools.partial(
        pl.kernel, mesh=mesh,
        out_type=(jax.ShapeDtypeStruct((R, DIM), F32),
                  jax.ShapeDtypeStruct((R, 128), F32)),
        scratch_types=[
            pltpu.VMEM((_CHUNK,), jnp.int32),
            pltpu.VMEM((_CHUNK,), jnp.int32),
            pltpu.VMEM((_CHUNK, 384), F32),
            pltpu.VMEM((_CHUNK, 384), F32),
            pltpu.SemaphoreType.DMA,
            pltpu.SemaphoreType.DMA,
        ],
    )
    def k(gidx_hbm, ktx_hbm, kg_hbm, xg_hbm, i0, i1, b0, b1, s0, s1):
        wid = lax.axis_index("s") * _NC + lax.axis_index("c")
        base = wid * _RPW
        ibufs = (i0, i1)
        bufs = (b0, b1)
        sems = (s0, s1)

        def load_and_fire(par, off):
            pltpu.sync_copy(gidx_hbm.at[pl.ds(off, _CHUNK)], ibufs[par])
            pltpu.async_copy(ktx_hbm.at[ibufs[par]], bufs[par], sems[par])

        def drain_and_store(par, off):
            pltpu.make_async_copy(ktx_hbm.at[ibufs[par]], bufs[par],
                                  sems[par]).wait()
            pltpu.sync_copy(bufs[par].at[:, pl.ds(0, DIM)],
                            kg_hbm.at[pl.ds(off, _CHUNK)])
            pltpu.sync_copy(bufs[par].at[:, pl.ds(DIM, 128)],
                            xg_hbm.at[pl.ds(off, _CHUNK)])

        load_and_fire(0, base)

        def body(i, carry):
            t = 2 * i
            off = base + t * _CHUNK
            load_and_fire(1, off + _CHUNK)
            drain_and_store(0, off)

            @pl.when(t + 2 < _NCHUNK)
            def _():
                load_and_fire(0, off + 2 * _CHUNK)

            drain_and_store(1, off + _CHUNK)
            return carry

        lax.fori_loop(0, _NCHUNK // 2, body, 0)

    return k(gidx, ktx)


def _k3_body(xg_ref, xt_ref, s1_ref, s2_ref):
    s = pl.program_id(0)
    nt = xt_ref.shape[0]
    center = xt_ref[...]
    rep = jnp.broadcast_to(center[:, None, :], (nt, K, 16))
    pcd = rep.reshape(nt * K, 16) - xg_ref[:, 0:16]

    @pl.when(s == 0)
    def _():
        s1_ref[...] = jnp.zeros_like(s1_ref)
        s2_ref[...] = jnp.zeros_like(s2_ref)

    s1_ref[...] += jnp.sum(pcd, axis=0, keepdims=True)
    s2_ref[...] += lax.dot_general(pcd, pcd, (((0,), (0,)), ((), ())),
                                   preferred_element_type=F32)


def _k3(xg, xt):
    steps = 16
    rows = R // steps
    nt = rows // K
    return pl.pallas_call(
        _k3_body,
        grid=(steps,),
        in_specs=[
            pl.BlockSpec((rows, 128), lambda s: (s, 0)),
            pl.BlockSpec((nt, 16), lambda s: (s, 0)),
        ],
        out_specs=[
            pl.BlockSpec((1, 16), lambda s: (0, 0)),
            pl.BlockSpec((16, 16), lambda s: (0, 0)),
        ],
        out_shape=[
            jax.ShapeDtypeStruct((1, 16), F32),
            jax.ShapeDtypeStruct((16, 16), F32),
        ],
    )(xg, xt)


_T4 = 256


def _k4_body(qt_ref, vt_ref, xt_ref, kg_ref, xg_ref, s1_ref, s2_ref,
             wp1_ref, bp1_ref, gp1_ref, bep1_ref, wp2_ref, bp2_ref,
             qk_ref, val_ref, sq1_ref, sq2_ref):
    s = pl.program_id(0)
    nt = _T4
    rows = nt * K

    center = xt_ref[...]
    rep = jnp.broadcast_to(center[:, None, :], (nt, K, 16))
    pcd = rep.reshape(rows, 16) - xg_ref[:, 0:16]

    w1 = jnp.concatenate([wp1_ref[...].T, jnp.zeros((13, 32), F32)], axis=0)
    h = jnp.dot(pcd, w1, preferred_element_type=F32) + bp1_ref[...][None, :]

    m_cnt = jnp.float32(R)
    mu = s1_ref[...] / m_cnt
    m_lin = jnp.dot(mu, w1, preferred_element_type=F32)
    t2 = jnp.dot(s2_ref[...] / m_cnt, w1, preferred_element_type=F32)
    diag = jnp.sum(w1 * t2, axis=0, keepdims=True)
    var = diag - m_lin * m_lin
    m_h = m_lin + bp1_ref[...][None, :]
    scale = gp1_ref[...][None, :] * lax.rsqrt(var + 1e-5)
    shift = bep1_ref[...][None, :] - m_h * scale
    hh = jnp.maximum(h * scale + shift, 0.0)

    pos = _mmT_bf(hh, wp2_ref[...]) + bp2_ref[...][None, :]

    q = qt_ref[...]
    v = vt_ref[...]
    q_rep = jnp.broadcast_to(q[:, None, :], (nt, K, DIM)).reshape(rows, DIM)
    v_rep = jnp.broadcast_to(v[:, None, :], (nt, K, DIM)).reshape(rows, DIM)
    kg = kg_ref[...]
    qk = q_rep - kg + pos
    val = v_rep - kg + pos
    qk8 = jnp.max(qk.reshape(rows // 2, 2, DIM), axis=1)
    val8 = jnp.max(val.reshape(rows // 2, 2, DIM), axis=1)
    qk_ref[...] = qk8.astype(BF)
    val_ref[...] = val8.astype(BF)

    @pl.when(s == 0)
    def _():
        sq1_ref[...] = jnp.zeros_like(sq1_ref)
        sq2_ref[...] = jnp.zeros_like(sq2_ref)

    sq1_ref[...] += jnp.sum(qk8, axis=0, keepdims=True)
    qk8b = qk8.astype(BF)
    sq2_ref[...] += lax.dot_general(qk8b, qk8b, (((0,), (0,)), ((), ())),
                                    preferred_element_type=F32)


def _k4(qt, vt, xt, kg, xg, s1, s2, wp1, bp1, gp1, bep1, wp2, bp2):
    steps = BN // _T4
    full = lambda shp: pl.BlockSpec(shp, lambda s: (0,) * len(shp))
    return pl.pallas_call(
        _k4_body,
        grid=(steps,),
        in_specs=[
            pl.BlockSpec((_T4, DIM), lambda s: (s, 0)),
            pl.BlockSpec((_T4, DIM), lambda s: (s, 0)),
            pl.BlockSpec((_T4, 16), lambda s: (s, 0)),
            pl.BlockSpec((_T4 * K, DIM), lambda s: (s, 0)),
            pl.BlockSpec((_T4 * K, 128), lambda s: (s, 0)),
            full((1, 16)), full((16, 16)),
            full((32, 3)), full((32,)), full((32,)), full((32,)),
            full((DIM, 32)), full((DIM,)),
        ],
        out_specs=[
            pl.BlockSpec((_T4 * K2, DIM), lambda s: (s, 0)),
            pl.BlockSpec((_T4 * K2, DIM), lambda s: (s, 0)),
            pl.BlockSpec((1, DIM), lambda s: (0, 0)),
            pl.BlockSpec((DIM, DIM), lambda s: (0, 0)),
        ],
        out_shape=[
            jax.ShapeDtypeStruct((BN * K2, DIM), BF),
            jax.ShapeDtypeStruct((BN * K2, DIM), BF),
            jax.ShapeDtypeStruct((1, DIM), F32),
            jax.ShapeDtypeStruct((DIM, DIM), F32),
        ],
    )(qt, vt, xt, kg, xg, s1, s2, wp1, bp1, gp1, bep1, wp2, bp2)


_T5 = 256


def _k5_body(qk_ref, val_ref, pt_ref, sq1_ref, sq2_ref,
             wa1_ref, ba1_ref, ga1_ref, bea1_ref, wa2_ref, ba2_ref,
             we_ref, be_ref, wup_ref, bup_ref, out_ref, ss_ref):
    s = pl.program_id(0)
    nt = _T5
    rows = nt * K2

    @pl.when(s == 0)
    def _():
        m_cnt = jnp.float32(BN * K2)
        mu = sq1_ref[...] / m_cnt
        cov = sq2_ref[...] / m_cnt - mu.reshape(DIM, 1) * mu
        wa1 = wa1_ref[...]
        t2 = jnp.dot(wa1, cov, preferred_element_type=F32)
        diag = jnp.sum(t2 * wa1, axis=1)
        m_lin = _mmT(mu, wa1)
        m_a = m_lin + ba1_ref[...][None, :]
        scale = ga1_ref[...][None, :] * lax.rsqrt(diag[None, :] + 1e-5)
        shift = bea1_ref[...][None, :] - m_a * scale
        ss_ref[0:1, :] = scale
        ss_ref[1:2, :] = shift

    scale = ss_ref[0:1, :]
    shift = ss_ref[1:2, :]
    a = _mmT_bf(qk_ref[...], wa1_ref[...]) + ba1_ref[...][None, :]
    a = jnp.maximum(a * scale + shift, 0.0)
    a2 = _mmT_bf(a, wa2_ref[...]) + ba2_ref[...][None, :]

    r = a2.reshape(nt, K2, DIM)
    m = jnp.max(r, axis=1, keepdims=True)
    e = jnp.exp(r - m)
    attn = e / jnp.sum(e, axis=1, keepdims=True)
    val = val_ref[...].astype(F32).reshape(nt, K2, DIM)
    agg = jnp.sum(attn * val, axis=1)

    y = _mmT_bf(agg, we_ref[...]) + be_ref[...][None, :] + pt_ref[...]
    out = _mmT_bf(y, wup_ref[...]) + bup_ref[...][None, :]
    out_ref[0] = out.T


def _k5(qk, val, pt, sq1, sq2, wa1, ba1, ga1, bea1, wa2, ba2,
        we, be, wup, bup):
    steps = BN // _T5
    npb = N // _T5
    full = lambda shp: pl.BlockSpec(shp, lambda s: (0,) * len(shp))
    return pl.pallas_call(
        _k5_body,
        grid=(steps,),
        in_specs=[
            pl.BlockSpec((_T5 * K2, DIM), lambda s: (s, 0)),
            pl.BlockSpec((_T5 * K2, DIM), lambda s: (s, 0)),
            pl.BlockSpec((_T5, CH), lambda s: (s, 0)),
            full((1, DIM)), full((DIM, DIM)),
            full((2 * DIM, DIM)), full((2 * DIM,)), full((2 * DIM,)),
            full((2 * DIM,)),
            full((DIM, 2 * DIM)), full((DIM,)),
            full((CH, DIM)), full((CH,)),
            full((CH, CH)), full((CH,)),
        ],
        out_specs=pl.BlockSpec((1, CH, _T5),
                               lambda s: (s // npb, 0, s % npb)),
        out_shape=jax.ShapeDtypeStruct((B, CH, N), F32),
        scratch_shapes=[pltpu.VMEM((2, 2 * DIM), F32)],
    )(qk, val, pt, sq1, sq2, wa1, ba1, ga1, bea1, wa2, ba2, we, be, wup, bup)


def kernel(xyz, points, Wq, bq, Wk, bk, Wv, bv, Wp1, bp1, gp1, betap1,
           Wp2, bp2, Wa1, ba1, ga1, betaa1, Wa2, ba2, We, be, Wup, bup):
    pt, qt, ktx, vt, xt, gidx = _k1(xyz, points, Wq, bq, Wk, bk, Wv, bv)
    kg, xg = _k2_sc_gather(gidx.reshape(-1), ktx)
    s1, s2 = _k3(xg, xt)
    qk, val, sq1, sq2 = _k4(qt, vt, xt, kg, xg, s1, s2,
                            Wp1, bp1, gp1, betap1, Wp2, bp2)
    new_points = _k5(qk, val, pt, sq1, sq2, Wa1, ba1, ga1, betaa1,
                     Wa2, ba2, We, be, Wup, bup)
    return (xyz, new_points)


# speedup vs baseline: 1.1207x; 1.1207x over previous
import functools

import jax
import jax.numpy as jnp
from jax import lax
from jax.experimental import pallas as pl
from jax.experimental.pallas import tpu as pltpu
from jax.experimental.pallas import tpu_sc as plsc

B = 4
N = 2048
CH = 256
DIM = 256
K = 16
K2 = 8
BN = B * N
R = B * N * K
F32 = jnp.float32


def _mmT(a, w):
    return lax.dot_general(a, w, (((1,), (1,)), ((), ())),
                           preferred_element_type=F32)


BF = jnp.bfloat16


def _mmT_bf(a, w):
    return lax.dot_general(a.astype(BF), w.astype(BF), (((1,), (1,)), ((), ())),
                           preferred_element_type=F32)


def _batcher_pairs(n):
    pairs = []
    p = 1
    while p < n:
        k = p
        while k >= 1:
            for j in range(k % p, n - k, 2 * k):
                for i in range(0, min(k, n - j - k)):
                    if (i + j) // (2 * p) == (i + j + k) // (2 * p):
                        pairs.append((i + j, i + j + k))
            k //= 2
        p *= 2
    return pairs

_SORT16 = _batcher_pairs(16)


def _k1_body(xyz_ref, pts_ref, wq_ref, bq_ref, wk_ref, bk_ref, wv_ref, bv_ref,
             pt_ref, qt_ref, ktx_ref, vt_ref, xt_ref, gidx_ref):
    b = pl.program_id(0)
    p = pts_ref[0]
    p_t = p.T
    pt_ref[...] = p_t
    qt_ref[...] = _mmT(p_t, wq_ref[...]) + bq_ref[...][None, :]
    k_t = _mmT(p_t, wk_ref[...]) + bk_ref[...][None, :]
    vt_ref[...] = _mmT(p_t, wv_ref[...]) + bv_ref[...][None, :]

    x = xyz_ref[0]
    x16 = jnp.concatenate([x, jnp.zeros((13, N), F32)], axis=0)
    x_t = x16.T
    xt_ref[...] = x_t
    ktx_ref[...] = jnp.zeros((N, 384), F32)
    ktx_ref[:, 0:DIM] = k_t
    ktx_ref[:, DIM:DIM + 16] = x_t
    sq = jnp.sum(x_t * x_t, axis=1)

    T = 512
    for t in range(N // T):
        xt_tile = x_t[t * T:(t + 1) * T]
        g = _mmT(xt_tile, x_t)
        d = sq[t * T:(t + 1) * T][:, None] + sq[None, :] - 2.0 * g
        iota = lax.broadcasted_iota(jnp.int32, (T, N), 1)
        idxs = []
        for _ in range(K):
            m = jnp.min(d, axis=1, keepdims=True)
            am = jnp.min(jnp.where(d == m, iota, N), axis=1)
            idxs.append(am)
            d = jnp.where(iota == am[:, None], jnp.inf, d)
        for (i, j) in _SORT16:
            lo = jnp.minimum(idxs[i], idxs[j])
            hi = jnp.maximum(idxs[i], idxs[j])
            idxs[i], idxs[j] = lo, hi
        tile_idx = jnp.stack(idxs, axis=1) + b * N
        gidx_ref[pl.ds(t * T, T), :] = tile_idx


def _k1(xyz, points, wq, bq, wk, bk, wv, bv):
    full = lambda s: pl.BlockSpec(s, lambda b: (0,) * len(s))
    flat = lambda c: pl.BlockSpec((N, c), lambda b: (b, 0))
    return pl.pallas_call(
        _k1_body,
        grid=(B,),
        in_specs=[
            pl.BlockSpec((1, 3, N), lambda b: (b, 0, 0)),
            pl.BlockSpec((1, CH, N), lambda b: (b, 0, 0)),
            full((DIM, CH)), full((DIM,)),
            full((DIM, CH)), full((DIM,)),
            full((DIM, CH)), full((DIM,)),
        ],
        out_specs=[flat(CH), flat(DIM), flat(384), flat(DIM), flat(16),
                   flat(K)],
        out_shape=[
            jax.ShapeDtypeStruct((BN, CH), F32),
            jax.ShapeDtypeStruct((BN, DIM), F32),
            jax.ShapeDtypeStruct((BN, 384), F32),
            jax.ShapeDtypeStruct((BN, DIM), F32),
            jax.ShapeDtypeStruct((BN, 16), F32),
            jax.ShapeDtypeStruct((BN, K), jnp.int32),
        ],
    )(xyz, points, wq, bq, wk, bk, wv, bv)


_NC = 2
_NS = 16
_NW = _NC * _NS
_RPW = R // _NW
_CHUNK = 128
_NCHUNK = _RPW // _CHUNK


def _k2_sc_gather(gidx, ktx):
    mesh = plsc.VectorSubcoreMesh(core_axis_name="c", subcore_axis_name="s")

    @functools.partial(
        pl.kernel, mesh=mesh,
        out_type=(jax.ShapeDtypeStruct((R, DIM), F32),
                  jax.ShapeDtypeStruct((R, 128), F32)),
        scratch_types=[
            pltpu.VMEM((_CHUNK,), jnp.int32),
            pltpu.VMEM((_CHUNK,), jnp.int32),
            pltpu.VMEM((_CHUNK, 384), F32),
            pltpu.VMEM((_CHUNK, 384), F32),
            pltpu.SemaphoreType.DMA,
            pltpu.SemaphoreType.DMA,
        ],
    )
    def k(gidx_hbm, ktx_hbm, kg_hbm, xg_hbm, i0, i1, b0, b1, s0, s1):
        wid = lax.axis_index("s") * _NC + lax.axis_index("c")
        base = wid * _RPW
        ibufs = (i0, i1)
        bufs = (b0, b1)
        sems = (s0, s1)

        def load_and_fire(par, off):
            pltpu.sync_copy(gidx_hbm.at[pl.ds(off, _CHUNK)], ibufs[par])
            pltpu.async_copy(ktx_hbm.at[ibufs[par]], bufs[par], sems[par])

        def drain_and_store(par, off):
            pltpu.make_async_copy(ktx_hbm.at[ibufs[par]], bufs[par],
                                  sems[par]).wait()
            pltpu.sync_copy(bufs[par].at[:, pl.ds(0, DIM)],
                            kg_hbm.at[pl.ds(off, _CHUNK)])
            pltpu.sync_copy(bufs[par].at[:, pl.ds(DIM, 128)],
                            xg_hbm.at[pl.ds(off, _CHUNK)])

        load_and_fire(0, base)

        def body(i, carry):
            t = 2 * i
            off = base + t * _CHUNK
            load_and_fire(1, off + _CHUNK)
            drain_and_store(0, off)

            @pl.when(t + 2 < _NCHUNK)
            def _():
                load_and_fire(0, off + 2 * _CHUNK)

            drain_and_store(1, off + _CHUNK)
            return carry

        lax.fori_loop(0, _NCHUNK // 2, body, 0)

    return k(gidx, ktx)


def _k3_body(xg_ref, xt_ref, s1_ref, s2_ref):
    s = pl.program_id(0)
    nt = xt_ref.shape[0]
    center = xt_ref[...]
    rep = jnp.broadcast_to(center[:, None, :], (nt, K, 16))
    pcd = rep.reshape(nt * K, 16) - xg_ref[:, 0:16]

    @pl.when(s == 0)
    def _():
        s1_ref[...] = jnp.zeros_like(s1_ref)
        s2_ref[...] = jnp.zeros_like(s2_ref)

    s1_ref[...] += jnp.sum(pcd, axis=0, keepdims=True)
    s2_ref[...] += lax.dot_general(pcd, pcd, (((0,), (0,)), ((), ())),
                                   preferred_element_type=F32)


def _k3(xg, xt):
    steps = 16
    rows = R // steps
    nt = rows // K
    return pl.pallas_call(
        _k3_body,
        grid=(steps,),
        in_specs=[
            pl.BlockSpec((rows, 128), lambda s: (s, 0)),
            pl.BlockSpec((nt, 16), lambda s: (s, 0)),
        ],
        out_specs=[
            pl.BlockSpec((1, 16), lambda s: (0, 0)),
            pl.BlockSpec((16, 16), lambda s: (0, 0)),
        ],
        out_shape=[
            jax.ShapeDtypeStruct((1, 16), F32),
            jax.ShapeDtypeStruct((16, 16), F32),
        ],
    )(xg, xt)


_T4 = 256


def _k4_body(qt_ref, vt_ref, xt_ref, kg_ref, xg_ref, s1_ref, s2_ref,
             wp1_ref, bp1_ref, gp1_ref, bep1_ref, wp2_ref, bp2_ref,
             qk_ref, val_ref, sq1_ref, sq2_ref):
    s = pl.program_id(0)
    nt = _T4
    rows = nt * K

    center = xt_ref[...]
    rep = jnp.broadcast_to(center[:, None, :], (nt, K, 16))
    pcd = rep.reshape(rows, 16) - xg_ref[:, 0:16]

    w1 = jnp.concatenate([wp1_ref[...].T, jnp.zeros((13, 32), F32)], axis=0)
    h = jnp.dot(pcd, w1, preferred_element_type=F32) + bp1_ref[...][None, :]

    m_cnt = jnp.float32(R)
    mu = s1_ref[...] / m_cnt
    m_lin = jnp.dot(mu, w1, preferred_element_type=F32)
    t2 = jnp.dot(s2_ref[...] / m_cnt, w1, preferred_element_type=F32)
    diag = jnp.sum(w1 * t2, axis=0, keepdims=True)
    var = diag - m_lin * m_lin
    m_h = m_lin + bp1_ref[...][None, :]
    scale = gp1_ref[...][None, :] * lax.rsqrt(var + 1e-5)
    shift = bep1_ref[...][None, :] - m_h * scale
    hh = jnp.maximum(h * scale + shift, 0.0)

    pos = _mmT(hh, wp2_ref[...]) + bp2_ref[...][None, :]

    q = qt_ref[...]
    v = vt_ref[...]
    q_rep = jnp.broadcast_to(q[:, None, :], (nt, K, DIM)).reshape(rows, DIM)
    v_rep = jnp.broadcast_to(v[:, None, :], (nt, K, DIM)).reshape(rows, DIM)
    kg = kg_ref[...]
    qk = q_rep - kg + pos
    val = v_rep - kg + pos
    qk8 = jnp.max(qk.reshape(rows // 2, 2, DIM), axis=1)
    val8 = jnp.max(val.reshape(rows // 2, 2, DIM), axis=1)
    qk_ref[...] = qk8
    val_ref[...] = val8

    @pl.when(s == 0)
    def _():
        sq1_ref[...] = jnp.zeros_like(sq1_ref)
        sq2_ref[...] = jnp.zeros_like(sq2_ref)

    sq1_ref[...] += jnp.sum(qk8, axis=0, keepdims=True)
    sq2_ref[...] += lax.dot_general(qk8, qk8, (((0,), (0,)), ((), ())),
                                    preferred_element_type=F32)


def _k4(qt, vt, xt, kg, xg, s1, s2, wp1, bp1, gp1, bep1, wp2, bp2):
    steps = BN // _T4
    full = lambda shp: pl.BlockSpec(shp, lambda s: (0,) * len(shp))
    return pl.pallas_call(
        _k4_body,
        grid=(steps,),
        in_specs=[
            pl.BlockSpec((_T4, DIM), lambda s: (s, 0)),
            pl.BlockSpec((_T4, DIM), lambda s: (s, 0)),
            pl.BlockSpec((_T4, 16), lambda s: (s, 0)),
            pl.BlockSpec((_T4 * K, DIM), lambda s: (s, 0)),
            pl.BlockSpec((_T4 * K, 128), lambda s: (s, 0)),
            full((1, 16)), full((16, 16)),
            full((32, 3)), full((32,)), full((32,)), full((32,)),
            full((DIM, 32)), full((DIM,)),
        ],
        out_specs=[
            pl.BlockSpec((_T4 * K2, DIM), lambda s: (s, 0)),
            pl.BlockSpec((_T4 * K2, DIM), lambda s: (s, 0)),
            pl.BlockSpec((1, DIM), lambda s: (0, 0)),
            pl.BlockSpec((DIM, DIM), lambda s: (0, 0)),
        ],
        out_shape=[
            jax.ShapeDtypeStruct((BN * K2, DIM), F32),
            jax.ShapeDtypeStruct((BN * K2, DIM), F32),
            jax.ShapeDtypeStruct((1, DIM), F32),
            jax.ShapeDtypeStruct((DIM, DIM), F32),
        ],
    )(qt, vt, xt, kg, xg, s1, s2, wp1, bp1, gp1, bep1, wp2, bp2)


_T5 = 256


def _k5_body(qk_ref, val_ref, pt_ref, sq1_ref, sq2_ref,
             wa1_ref, ba1_ref, ga1_ref, bea1_ref, wa2_ref, ba2_ref,
             we_ref, be_ref, wup_ref, bup_ref, out_ref, ss_ref):
    s = pl.program_id(0)
    nt = _T5
    rows = nt * K2

    @pl.when(s == 0)
    def _():
        m_cnt = jnp.float32(BN * K2)
        mu = sq1_ref[...] / m_cnt
        cov = sq2_ref[...] / m_cnt - mu.reshape(DIM, 1) * mu
        wa1 = wa1_ref[...]
        t2 = jnp.dot(wa1, cov, preferred_element_type=F32)
        diag = jnp.sum(t2 * wa1, axis=1)
        m_lin = _mmT(mu, wa1)
        m_a = m_lin + ba1_ref[...][None, :]
        scale = ga1_ref[...][None, :] * lax.rsqrt(diag[None, :] + 1e-5)
        shift = bea1_ref[...][None, :] - m_a * scale
        ss_ref[0:1, :] = scale
        ss_ref[1:2, :] = shift

    scale = ss_ref[0:1, :]
    shift = ss_ref[1:2, :]
    a = _mmT(qk_ref[...], wa1_ref[...]) + ba1_ref[...][None, :]
    a = jnp.maximum(a * scale + shift, 0.0)
    a2 = _mmT(a, wa2_ref[...]) + ba2_ref[...][None, :]

    r = a2.reshape(nt, K2, DIM)
    m = jnp.max(r, axis=1, keepdims=True)
    e = jnp.exp(r - m)
    attn = e / jnp.sum(e, axis=1, keepdims=True)
    val = val_ref[...].reshape(nt, K2, DIM)
    agg = jnp.sum(attn * val, axis=1)

    y = _mmT(agg, we_ref[...]) + be_ref[...][None, :] + pt_ref[...]
    out = _mmT(y, wup_ref[...]) + bup_ref[...][None, :]
    out_ref[0] = out.T


def _k5(qk, val, pt, sq1, sq2, wa1, ba1, ga1, bea1, wa2, ba2,
        we, be, wup, bup):
    steps = BN // _T5
    npb = N // _T5
    full = lambda shp: pl.BlockSpec(shp, lambda s: (0,) * len(shp))
    return pl.pallas_call(
        _k5_body,
        grid=(steps,),
        in_specs=[
            pl.BlockSpec((_T5 * K2, DIM), lambda s: (s, 0)),
            pl.BlockSpec((_T5 * K2, DIM), lambda s: (s, 0)),
            pl.BlockSpec((_T5, CH), lambda s: (s, 0)),
            full((1, DIM)), full((DIM, DIM)),
            full((2 * DIM, DIM)), full((2 * DIM,)), full((2 * DIM,)),
            full((2 * DIM,)),
            full((DIM, 2 * DIM)), full((DIM,)),
            full((CH, DIM)), full((CH,)),
            full((CH, CH)), full((CH,)),
        ],
        out_specs=pl.BlockSpec((1, CH, _T5),
                               lambda s: (s // npb, 0, s % npb)),
        out_shape=jax.ShapeDtypeStruct((B, CH, N), F32),
        scratch_shapes=[pltpu.VMEM((2, 2 * DIM), F32)],
    )(qk, val, pt, sq1, sq2, wa1, ba1, ga1, bea1, wa2, ba2, we, be, wup, bup)


def kernel(xyz, points, Wq, bq, Wk, bk, Wv, bv, Wp1, bp1, gp1, betap1,
           Wp2, bp2, Wa1, ba1, ga1, betaa1, Wa2, ba2, We, be, Wup, bup):
    pt, qt, ktx, vt, xt, gidx = _k1(xyz, points, Wq, bq, Wk, bk, Wv, bv)
    kg, xg = _k2_sc_gather(gidx.reshape(-1), ktx)
    s1, s2 = _k3(xg, xt)
    qk, val, sq1, sq2 = _k4(qt, vt, xt, kg, xg, s1, s2,
                            Wp1, bp1, gp1, betap1, Wp2, bp2)
    new_points = _k5(qk, val, pt, sq1, sq2, Wa1, ba1, ga1, betaa1,
                     Wa2, ba2, We, be, Wup, bup)
    return (xyz, new_points)
